# baseline (speedup 1.0000x reference)
import dataclasses
import functools

import jax
import jax.numpy as jnp
from jax import lax
from jax.experimental import pallas as pl
from jax.experimental.pallas import tpu as pltpu
from jax.experimental.pallas import tpu_sc as plsc

E = 320000
N = 10000
D = 128
NC = 2
NS = 16
NW = NC * NS
EW = E // NW
C = 80
NCH = EW // C
G = C // 16
L = 16


def _rsqrt(x):
    i = lax.bitcast_convert_type(x, jnp.int32)
    i = jnp.int32(0x5F3759DF) - (i >> 1)
    y = lax.bitcast_convert_type(i, jnp.float32)
    for _ in range(3):
        y = y * (1.5 - 0.5 * x * y * y)
    return y


def _norms_body(z_ref, ss_ref):
    z = z_ref[...]
    ss_ref[...] = jnp.sum(z * z, axis=1, keepdims=True)


_norms_tc = pl.pallas_call(
    _norms_body,
    out_shape=jax.ShapeDtypeStruct((N, 1), jnp.float32),
)

_mesh = plsc.VectorSubcoreMesh(core_axis_name="c", subcore_axis_name="s")

_cp = pltpu.CompilerParams()
if "needs_layout_passes" in pltpu.CompilerParams.__dataclass_fields__:
    _cp = dataclasses.replace(_cp, needs_layout_passes=False)


@functools.partial(
    pl.kernel,
    mesh=_mesh,
    compiler_params=_cp,
    out_type=jax.ShapeDtypeStruct((E,), jnp.float32),
    scratch_types=[
        pltpu.VMEM_SHARED((N, D), jnp.float32),
        pltpu.VMEM((2, C), jnp.int32),
        pltpu.VMEM((2, C), jnp.int32),
        pltpu.VMEM((2, C), jnp.float32),
        pltpu.VMEM((2, C), jnp.float32),
        pltpu.VMEM((2, C), jnp.float32),
        pltpu.VMEM((C, D), jnp.float32),
        pltpu.VMEM((C, D), jnp.float32),
        pltpu.VMEM((C, D), jnp.float32),
        pltpu.VMEM((C, D), jnp.float32),
        pltpu.SemaphoreType.DMA,
        pltpu.SemaphoreType.DMA,
        pltpu.SemaphoreType.DMA,
        pltpu.SemaphoreType.DMA,
        pltpu.SemaphoreType.DMA,
        pltpu.SemaphoreType.DMA,
        pltpu.SemaphoreType.DMA,
        pltpu.SemaphoreType.DMA,
        pltpu.SemaphoreType.DMA,
        pltpu.SemaphoreType.DMA,
        pltpu.SemaphoreType.DMA,
        pltpu.SemaphoreType.DMA,
        pltpu.SemaphoreType.DMA,
        pltpu.SemaphoreType.DMA,
    ],
)
def _cosine_sc(z_hbm, src_hbm, dst_hbm, ss_hbm, out_hbm,
               zsp, sidx, didx, ssb, ddb, outb,
               srA, drA, srB, drB,
               semsi0, semsi1, semdi0, semdi1,
               semsr0, semsr1, semdr0, semdr1,
               semsn0, semsn1, semdn0, semdn1,
               semo0, semo1):
    sid = lax.axis_index("s")
    wid = sid * NC + lax.axis_index("c")
    base = wid * EW
    rbufs = ((srA, drA), (srB, drB))
    semsi = (semsi0, semsi1)
    semdi = (semdi0, semdi1)
    semsr = (semsr0, semsr1)
    semdr = (semdr0, semdr1)
    semsn = (semsn0, semsn1)
    semdn = (semdn0, semdn1)
    semo = (semo0, semo1)

    rows_per_tile = 624
    pltpu.sync_copy(z_hbm.at[pl.ds(sid * rows_per_tile, rows_per_tile)],
                    zsp.at[pl.ds(sid * rows_per_tile, rows_per_tile)])

    @pl.when(sid == 0)
    def _tail():
        pltpu.sync_copy(z_hbm.at[pl.ds(NS * rows_per_tile, N - NS * rows_per_tile)],
                        zsp.at[pl.ds(NS * rows_per_tile, N - NS * rows_per_tile)])

    plsc.subcore_barrier()

    def idx_copies(ci, b):
        off = base + ci * C
        return (
            pltpu.make_async_copy(src_hbm.at[pl.ds(off, C)], sidx.at[b], semsi[b]),
            pltpu.make_async_copy(dst_hbm.at[pl.ds(off, C)], didx.at[b], semdi[b]),
        )

    def gather_copies(ci, b):
        sr, dr = rbufs[b]
        return (
            pltpu.make_async_copy(zsp.at[sidx.at[b]], sr, semsr[b]),
            pltpu.make_async_copy(z_hbm.at[didx.at[b]], dr, semdr[b]),
            pltpu.make_async_copy(ss_hbm.at[sidx.at[b]], ssb.at[b], semsn[b]),
            pltpu.make_async_copy(ss_hbm.at[didx.at[b]], ddb.at[b], semdn[b]),
        )

    def out_copy(ci, b):
        off = base + ci * C
        return pltpu.make_async_copy(outb.at[b], out_hbm.at[pl.ds(off, C)],
                                     semo[b])

    def start(copies):
        for cp in copies:
            cp.start()

    def wait(copies):
        for cp in copies:
            cp.wait()

    def compute(ci, b):
        sr, dr = rbufs[b]
        for g in range(G):
            e0 = g * L
            erow = lax.iota(jnp.int32, L) + e0
            lane = lax.iota(jnp.int32, L)
            zero = jnp.zeros((L,), jnp.float32)

            def fbody(f, dotv):
                fv = (lane + f) & (D - 1)
                s = plsc.load_gather(sr, [erow, fv])
                d = plsc.load_gather(dr, [erow, fv])
                return dotv + s * d

            dotv = lax.fori_loop(0, D, fbody, zero, unroll=8)
            ssv = ssb[b, pl.ds(e0, L)]
            ddv = ddb[b, pl.ds(e0, L)]
            prod = jnp.maximum(ssv * ddv, 1e-12)
            val = dotv * _rsqrt(prod)
            sig = 1.0 / (1.0 + jnp.exp(-val))
            outb[b, pl.ds(e0, L)] = sig

    start(idx_copies(0, 0))
    start(idx_copies(1, 1))
    wait(idx_copies(0, 0))
    start(gather_copies(0, 0))

    @pl.loop(0, NCH, step=2)
    def _pair(i):
        def step(ci, b):
            @pl.when(ci + 1 < NCH)
            def _():
                wait(idx_copies(ci + 1, 1 - b))
                start(gather_copies(ci + 1, 1 - b))

            wait(gather_copies(ci, b))

            @pl.when(ci >= 2)
            def _():
                wait([out_copy(ci - 2, b)])

            compute(ci, b)
            start([out_copy(ci, b)])

            @pl.when(ci + 2 < NCH)
            def _():
                start(idx_copies(ci + 2, b))

        step(i, 0)

        @pl.when(i + 1 < NCH)
        def _():
            step(i + 1, 1)

    wait([out_copy(NCH - 2, 0 if (NCH - 2) % 2 == 0 else 1)])
    wait([out_copy(NCH - 1, 0 if (NCH - 1) % 2 == 0 else 1)])


def kernel(z, edge_index):
    ei = edge_index.astype(jnp.int32)
    ss = _norms_tc(z).reshape(N)
    return _cosine_sc(z, ei[0], ei[1], ss)


# speedup vs baseline: 1.0809x; 1.0809x over previous
import dataclasses
import functools

import jax
import jax.numpy as jnp
from jax import lax
from jax.experimental import pallas as pl
from jax.experimental.pallas import tpu as pltpu
from jax.experimental.pallas import tpu_sc as plsc

E = 320000
N = 10000
D = 128
P = D // 2
NC = 2
NS = 16
NW = NC * NS
EW = E // NW
C = 80
NCH = EW // C
G = C // 16
L = 16


def _rsqrt(x):
    i = lax.bitcast_convert_type(x, jnp.int32)
    i = jnp.int32(0x5F3759DF) - (i >> 1)
    y = lax.bitcast_convert_type(i, jnp.float32)
    for _ in range(3):
        y = y * (1.5 - 0.5 * x * y * y)
    return y


def _norms_body(z_ref, ss_ref):
    z = z_ref[...]
    ss_ref[...] = jnp.sum(z * z, axis=1, keepdims=True)


_norms_tc = pl.pallas_call(
    _norms_body,
    out_shape=jax.ShapeDtypeStruct((N, 1), jnp.float32),
)

_mesh = plsc.VectorSubcoreMesh(core_axis_name="c", subcore_axis_name="s")

_cp = pltpu.CompilerParams()
if "needs_layout_passes" in pltpu.CompilerParams.__dataclass_fields__:
    _cp = dataclasses.replace(_cp, needs_layout_passes=False)
if "use_tc_tiling_on_sc" in pltpu.CompilerParams.__dataclass_fields__:
    _cp = dataclasses.replace(_cp, use_tc_tiling_on_sc=False)


@functools.partial(
    pl.kernel,
    mesh=_mesh,
    compiler_params=_cp,
    out_type=jax.ShapeDtypeStruct((E,), jnp.float32),
    scratch_types=[
        pltpu.VMEM((EW,), jnp.int32),
        pltpu.VMEM((EW,), jnp.int32),
        pltpu.VMEM((EW,), jnp.float32),
        pltpu.VMEM((N,), jnp.float32),
        pltpu.VMEM((C, P), jnp.int32),
        pltpu.VMEM((C, P), jnp.int32),
        pltpu.VMEM((C, P), jnp.int32),
        pltpu.VMEM((C, P), jnp.int32),
        pltpu.SemaphoreType.DMA,
        pltpu.SemaphoreType.DMA,
        pltpu.SemaphoreType.DMA,
        pltpu.SemaphoreType.DMA,
    ],
)
def _cosine_sc(zp_hbm, src_hbm, dst_hbm, ss_hbm, out_hbm,
               sidx, didx, outv, ssn, srA, drA, srB, drB,
               ssA, sdA, ssB, sdB):
    wid = lax.axis_index("s") * NC + lax.axis_index("c")
    base = wid * EW
    bufs = ((srA, drA, ssA, sdA), (srB, drB, ssB, sdB))

    pltpu.sync_copy(src_hbm.at[pl.ds(base, EW)], sidx)
    pltpu.sync_copy(dst_hbm.at[pl.ds(base, EW)], didx)
    pltpu.sync_copy(ss_hbm, ssn)

    def start(ci, b):
        sr, dr, ss, sd = bufs[b]
        pltpu.async_copy(zp_hbm.at[sidx.at[pl.ds(ci * C, C)]], sr, ss)
        pltpu.async_copy(zp_hbm.at[didx.at[pl.ds(ci * C, C)]], dr, sd)

    def wait(ci, b):
        sr, dr, ss, sd = bufs[b]
        pltpu.make_async_copy(zp_hbm.at[sidx.at[pl.ds(ci * C, C)]], sr, ss).wait()
        pltpu.make_async_copy(zp_hbm.at[didx.at[pl.ds(ci * C, C)]], dr, sd).wait()

    def compute(ci, b):
        sr, dr, _, _ = bufs[b]
        for g in range(G):
            e0 = g * L
            erow = lax.iota(jnp.int32, L) + e0
            lane = lax.iota(jnp.int32, L)
            zero = jnp.zeros((L,), jnp.float32)

            def fbody(k, dotv):
                kv = (lane + k) & (P - 1)
                sp = plsc.load_gather(sr, [erow, kv])
                dp = plsc.load_gather(dr, [erow, kv])
                sa, sb = plsc.unpack(plsc.bitcast(sp, jnp.bfloat16),
                                     format=plsc.PackFormat.INTERLEAVED)
                da, db = plsc.unpack(plsc.bitcast(dp, jnp.bfloat16),
                                     format=plsc.PackFormat.INTERLEAVED)
                return dotv + (sa * da + sb * db)

            dotv = lax.fori_loop(0, P, fbody, zero, unroll=8)
            snod = sidx[pl.ds(ci * C + e0, L)]
            dnod = didx[pl.ds(ci * C + e0, L)]
            ssv = plsc.load_gather(ssn, [snod])
            ddv = plsc.load_gather(ssn, [dnod])
            prod = jnp.maximum(ssv * ddv, 1e-12)
            val = dotv * _rsqrt(prod)
            sig = 1.0 / (1.0 + jnp.exp(-val))
            outv[pl.ds(ci * C + e0, L)] = sig

    start(0, 0)
    start(1, 1)

    @pl.loop(0, NCH, step=2)
    def _pair(i):
        def step(ci, b):
            wait(ci, b)
            compute(ci, b)

            @pl.when(ci + 2 < NCH)
            def _():
                start(ci + 2, b)

        step(i, 0)

        @pl.when(i + 1 < NCH)
        def _():
            step(i + 1, 1)

    pltpu.sync_copy(outv, out_hbm.at[pl.ds(base, EW)])


def kernel(z, edge_index):
    ei = edge_index.astype(jnp.int32)
    zb = z.astype(jnp.bfloat16)
    zp = lax.bitcast_convert_type(zb.reshape(N, P, 2), jnp.int32)
    ss = _norms_tc(zb.astype(jnp.float32)).reshape(N)
    return _cosine_sc(zp, ei[0], ei[1], ss)


# speedup vs baseline: 1.1646x; 1.0774x over previous
import dataclasses
import functools

import jax
import jax.numpy as jnp
from jax import lax
from jax.experimental import pallas as pl
from jax.experimental.pallas import tpu as pltpu
from jax.experimental.pallas import tpu_sc as plsc

E = 320000
N = 10000
D = 128
P = D // 2
NC = 2
NS = 16
NW = NC * NS
EW = E // NW
C = 80
NCH = EW // C
G = C // 16
L = 16


def _rsqrt(x):
    i = lax.bitcast_convert_type(x, jnp.int32)
    i = jnp.int32(0x5F3759DF) - (i >> 1)
    y = lax.bitcast_convert_type(i, jnp.float32)
    for _ in range(3):
        y = y * (1.5 - 0.5 * x * y * y)
    return y


def _norms_body(z_ref, ss_ref):
    z = z_ref[...]
    ss_ref[...] = jnp.sum(z * z, axis=1, keepdims=True)


_norms_tc = pl.pallas_call(
    _norms_body,
    out_shape=jax.ShapeDtypeStruct((N, 1), jnp.float32),
)

_mesh = plsc.VectorSubcoreMesh(core_axis_name="c", subcore_axis_name="s")

_cp = pltpu.CompilerParams()
if "needs_layout_passes" in pltpu.CompilerParams.__dataclass_fields__:
    _cp = dataclasses.replace(_cp, needs_layout_passes=False)
if "use_tc_tiling_on_sc" in pltpu.CompilerParams.__dataclass_fields__:
    _cp = dataclasses.replace(_cp, use_tc_tiling_on_sc=False)


@functools.partial(
    pl.kernel,
    mesh=_mesh,
    compiler_params=_cp,
    out_type=jax.ShapeDtypeStruct((E,), jnp.float32),
    scratch_types=[
        pltpu.VMEM((EW,), jnp.int32),
        pltpu.VMEM((EW,), jnp.int32),
        pltpu.VMEM((EW,), jnp.float32),
        pltpu.VMEM((N,), jnp.float32),
        pltpu.VMEM((C, P), jnp.int32),
        pltpu.VMEM((C, P), jnp.int32),
        pltpu.VMEM((C, P), jnp.int32),
        pltpu.VMEM((C, P), jnp.int32),
        pltpu.SemaphoreType.DMA,
        pltpu.SemaphoreType.DMA,
        pltpu.SemaphoreType.DMA,
        pltpu.SemaphoreType.DMA,
    ],
)
def _cosine_sc(zp_hbm, src_hbm, dst_hbm, ss_hbm, out_hbm,
               sidx, didx, outv, ssn, srA, drA, srB, drB,
               ssA, sdA, ssB, sdB):
    wid = lax.axis_index("s") * NC + lax.axis_index("c")
    base = wid * EW
    bufs = ((srA, drA, ssA, sdA), (srB, drB, ssB, sdB))

    pltpu.sync_copy(src_hbm.at[pl.ds(base, EW)], sidx)
    pltpu.sync_copy(dst_hbm.at[pl.ds(base, EW)], didx)
    pltpu.sync_copy(ss_hbm, ssn)

    def start(ci, b):
        sr, dr, ss, sd = bufs[b]
        pltpu.async_copy(zp_hbm.at[sidx.at[pl.ds(ci * C, C)]], sr, ss)
        pltpu.async_copy(zp_hbm.at[didx.at[pl.ds(ci * C, C)]], dr, sd)

    def wait(ci, b):
        sr, dr, ss, sd = bufs[b]
        pltpu.make_async_copy(zp_hbm.at[sidx.at[pl.ds(ci * C, C)]], sr, ss).wait()
        pltpu.make_async_copy(zp_hbm.at[didx.at[pl.ds(ci * C, C)]], dr, sd).wait()

    def compute(ci, b):
        sr, dr, _, _ = bufs[b]
        for g in range(G):
            e0 = g * L
            erow = lax.iota(jnp.int32, L) + e0
            lane = lax.iota(jnp.int32, L)
            zero = jnp.zeros((L,), jnp.float32)

            def fbody(m, dotv):
                kv1 = (lane + 2 * m) & (P - 1)
                kv2 = (lane + 2 * m + 1) & (P - 1)
                sp1 = plsc.load_gather(sr, [erow, kv1])
                dp1 = plsc.load_gather(dr, [erow, kv1])
                sp2 = plsc.load_gather(sr, [erow, kv2])
                dp2 = plsc.load_gather(dr, [erow, kv2])
                m1 = plsc.bitcast(sp1, jnp.bfloat16) * plsc.bitcast(dp1, jnp.bfloat16)
                m2 = plsc.bitcast(sp2, jnp.bfloat16) * plsc.bitcast(dp2, jnp.bfloat16)
                ps = plsc.bitcast(m1 + m2, jnp.int32)
                lo = lax.bitcast_convert_type(ps << 16, jnp.float32)
                hi = lax.bitcast_convert_type(ps & jnp.int32(-65536), jnp.float32)
                return dotv + (lo + hi)

            dotv = lax.fori_loop(0, P // 2, fbody, zero, unroll=8)
            snod = sidx[pl.ds(ci * C + e0, L)]
            dnod = didx[pl.ds(ci * C + e0, L)]
            ssv = plsc.load_gather(ssn, [snod])
            ddv = plsc.load_gather(ssn, [dnod])
            prod = jnp.maximum(ssv * ddv, 1e-12)
            val = dotv * _rsqrt(prod)
            sig = 1.0 / (1.0 + jnp.exp(-val))
            outv[pl.ds(ci * C + e0, L)] = sig

    start(0, 0)
    start(1, 1)

    @pl.loop(0, NCH, step=2)
    def _pair(i):
        def step(ci, b):
            wait(ci, b)
            compute(ci, b)

            @pl.when(ci + 2 < NCH)
            def _():
                start(ci + 2, b)

        step(i, 0)

        @pl.when(i + 1 < NCH)
        def _():
            step(i + 1, 1)

    pltpu.sync_copy(outv, out_hbm.at[pl.ds(base, EW)])


def kernel(z, edge_index):
    ei = edge_index.astype(jnp.int32)
    zb = z.astype(jnp.bfloat16)
    zp = lax.bitcast_convert_type(zb.reshape(N, P, 2), jnp.int32)
    ss = _norms_tc(zb.astype(jnp.float32)).reshape(N)
    return _cosine_sc(zp, ei[0], ei[1], ss)
